# Initial kernel scaffold; baseline (speedup 1.0000x reference)
import jax
import jax.numpy as jnp
from jax.experimental import pallas as pl


def kernel(cloud, indices, params):
    raise NotImplementedError("write your pallas kernel here")


# speedup vs baseline: 1.5299x; 1.5299x over previous
import functools

import jax
import jax.numpy as jnp
from jax import lax
from jax.experimental import pallas as pl
from jax.experimental.pallas import tpu as pltpu

KK = 20
H_DIMS = [64, 64, 128, 256]
EMB_DIM = 1024
NEG_SLOPE = 0.2


def _leaky(x):
    return jnp.where(x >= 0, x, NEG_SLOPE * x)


def _bf(x):
    return x.astype(jnp.bfloat16)


def _topk_body(xt_ref, xf_ref, xxc_ref, xxr_ref, idx_ref, *, T, N):
    xt = xt_ref[...]
    xf = xf_ref[...]
    s = lax.dot_general(_bf(xt), _bf(xf), (((1,), (1,)), ((), ())),
                        preferred_element_type=jnp.float32)
    d = 2.0 * s - xxr_ref[0] - xxc_ref[...]
    iota = lax.broadcasted_iota(jnp.int32, (T, N), 1)
    for j in range(KK):
        m = jnp.max(d, axis=1, keepdims=True)
        am = jnp.min(jnp.where(d >= m, iota, N), axis=1)
        idx_ref[0, j, :] = am
        d = jnp.where(iota == am[:, None], -jnp.inf, d)


def _topk(xt, xx, B, N, T):
    C = xt.shape[1]
    R = N // T
    body = functools.partial(_topk_body, T=T, N=N)
    return pl.pallas_call(
        body,
        grid=(B, R),
        in_specs=[
            pl.BlockSpec((T, C), lambda b, r: (b * R + r, 0)),
            pl.BlockSpec((N, C), lambda b, r: (b, 0)),
            pl.BlockSpec((T, 1), lambda b, r: (b * R + r, 0)),
            pl.BlockSpec((1, 1, N), lambda b, r: (b, 0, 0)),
        ],
        out_specs=pl.BlockSpec((1, KK, T), lambda b, r: (b, 0, r)),
        out_shape=jax.ShapeDtypeStruct((B, KK, N), jnp.int32),
    )(xt, xt, xx, xx.reshape(B, 1, N))


def _edgeconv_body(e_ref, x_ref, w1_ref, w2_ref, b_ref, g_ref, bt_ref,
                   out_ref, xx_ref, a_s):
    k = pl.program_id(1)

    @pl.when(k == 0)
    def _():
        a_s[...] = lax.dot_general(
            _bf(x_ref[...]), _bf(w2_ref[...]), (((1,), (0,)), ((), ())),
            preferred_element_type=jnp.float32) + b_ref[...]

    h = lax.dot_general(e_ref[...], _bf(w1_ref[...]), (((1,), (0,)), ((), ())),
                        preferred_element_type=jnp.float32)
    v = _leaky(g_ref[...] * (a_s[...] + h) + bt_ref[...])
    prev = jnp.where(k == 0, -jnp.inf, out_ref[...])
    m = jnp.maximum(prev, v)
    out_ref[...] = m

    @pl.when(k == KK - 1)
    def _():
        xx_ref[...] = jnp.sum(m * m, axis=1, keepdims=True)


def _edgeconv(E, X, w1t, w2t, b, g, bt, BN, T):
    C = X.shape[1]
    do = w1t.shape[1]
    Tn = BN // T
    return pl.pallas_call(
        _edgeconv_body,
        grid=(Tn, KK),
        in_specs=[
            pl.BlockSpec((T, C), lambda t, k: (k * Tn + t, 0)),
            pl.BlockSpec((T, C), lambda t, k: (t, 0)),
            pl.BlockSpec((C, do), lambda t, k: (0, 0)),
            pl.BlockSpec((C, do), lambda t, k: (0, 0)),
            pl.BlockSpec((1, do), lambda t, k: (0, 0)),
            pl.BlockSpec((1, do), lambda t, k: (0, 0)),
            pl.BlockSpec((1, do), lambda t, k: (0, 0)),
        ],
        out_specs=[
            pl.BlockSpec((T, do), lambda t, k: (t, 0)),
            pl.BlockSpec((T, 1), lambda t, k: (t, 0)),
        ],
        out_shape=[
            jax.ShapeDtypeStruct((BN, do), jnp.float32),
            jax.ShapeDtypeStruct((BN, 1), jnp.float32),
        ],
        scratch_shapes=[pltpu.VMEM((T, do), jnp.float32)],
    )(E, X, w1t, w2t, b, g, bt)


def _head_body(x1_ref, x2_ref, x3_ref, x4_ref, w1_ref, w2_ref, w3_ref, w4_ref,
               b_ref, g_ref, bt_ref, o_ref):
    r = pl.program_id(1)

    def dot(xr, wr):
        return lax.dot_general(_bf(xr[...]), _bf(wr[...]),
                               (((1,), (0,)), ((), ())),
                               preferred_element_type=jnp.float32)

    h = dot(x1_ref, w1_ref) + dot(x2_ref, w2_ref)
    h = h + dot(x3_ref, w3_ref) + dot(x4_ref, w4_ref)
    h = _leaky(g_ref[...] * (h + b_ref[...]) + bt_ref[...])
    pmax = jnp.max(h, axis=0, keepdims=True)
    psum = jnp.sum(h, axis=0, keepdims=True)

    @pl.when(r == 0)
    def _():
        o_ref[0, :, :] = jnp.concatenate([pmax, psum], axis=0)

    @pl.when(r > 0)
    def _():
        o_ref[0, 0, :] = jnp.maximum(o_ref[0, 0, :], pmax[0])
        o_ref[0, 1, :] = o_ref[0, 1, :] + psum[0]


def _head(xs, wfs, fb, fg, fbt, B, N, T):
    R = N // T
    in_specs = []
    args = []
    for x in xs:
        d = x.shape[1]
        in_specs.append(pl.BlockSpec((T, d), lambda b, r, R=R: (b * R + r, 0)))
        args.append(x)
    for w in wfs:
        d = w.shape[0]
        in_specs.append(pl.BlockSpec((d, EMB_DIM), lambda b, r: (0, 0)))
        args.append(w)
    for v in (fb, fg, fbt):
        in_specs.append(pl.BlockSpec((1, EMB_DIM), lambda b, r: (0, 0)))
        args.append(v)
    return pl.pallas_call(
        _head_body,
        grid=(B, R),
        in_specs=in_specs,
        out_specs=pl.BlockSpec((1, 2, EMB_DIM), lambda b, r: (b, 0, 0)),
        out_shape=jax.ShapeDtypeStruct((B, 2, EMB_DIM), jnp.float32),
    )(*args)


def _mlp_body(x_ref, w0_ref, b0_ref, g0_ref, t0_ref, w1_ref, b1_ref, g1_ref,
              t1_ref, w2_ref, b2_ref, o_ref):
    def dot(x, wr):
        return lax.dot_general(_bf(x), _bf(wr[...]), (((1,), (0,)), ((), ())),
                               preferred_element_type=jnp.float32)

    h = _leaky(g0_ref[...] * (dot(x_ref[...], w0_ref) + b0_ref[...]) + t0_ref[...])
    h = _leaky(g1_ref[...] * (dot(h, w1_ref) + b1_ref[...]) + t1_ref[...])
    o_ref[...] = dot(h, w2_ref) + b2_ref[...]


def _mlp(x, args):
    B = x.shape[0]
    n_out = args[-2].shape[1]
    return pl.pallas_call(
        _mlp_body,
        out_shape=jax.ShapeDtypeStruct((B, n_out), jnp.float32),
    )(x, *args)


def _gather_diff(X, idx_tr, B, N):
    C = X.shape[1]
    Xr = X.reshape(B, N, C)
    nbr = jax.vmap(lambda xb, ib: xb[ib])(Xr, idx_tr)
    E = (nbr - Xr[:, None, :, :]).astype(jnp.bfloat16)
    return jnp.transpose(E, (1, 0, 2, 3)).reshape(KK * B * N, C)


def kernel(cloud, indices, params):
    B, N, _ = cloud.shape
    BN = B * N
    xt = cloud.reshape(BN, 3)
    xx = None
    idx_tr = jnp.transpose(indices, (0, 2, 1))

    xs = []
    for i in range(4):
        C = xt.shape[1]
        do = H_DIMS[i]
        if i > 0:
            idx_tr = _topk(xt, xx, B, N, T=256)
        E = _gather_diff(xt, idx_tr, B, N)
        w = params['conv_w_%d' % i]
        xt, xx = _edgeconv(
            E, xt, w[:, :C].T, w[:, C:].T,
            params['conv_b_%d' % i][None, :], params['conv_g_%d' % i][None, :],
            params['conv_bt_%d' % i][None, :], BN, T=512)
        xs.append(xt)

    offs = [0, 64, 128, 256, 512]
    fw = params['final_w']
    wfs = [fw[:, offs[i]:offs[i + 1]].T for i in range(4)]
    pooled = _head(xs, wfs, params['final_b'][None, :], params['final_g'][None, :],
                   params['final_bt'][None, :], B, N, T=512)
    emb = jnp.concatenate([pooled[:, 0, :], pooled[:, 1, :] / N], axis=1)

    margs = []
    for j in range(2):
        margs += [params['mlp_w_%d' % j].T, params['mlp_b_%d' % j][None, :],
                  params['mlp_g_%d' % j][None, :], params['mlp_bt_%d' % j][None, :]]
    margs += [params['out_w'].T, params['out_b'][None, :]]
    return _mlp(emb, margs)


# speedup vs baseline: 7.3950x; 4.8337x over previous
import functools

import jax
import jax.numpy as jnp
from jax import lax
from jax.experimental import pallas as pl
from jax.experimental.pallas import tpu as pltpu
from jax.experimental.pallas import tpu_sc as plsc

KK = 20
H_DIMS = [64, 64, 128, 256]
EMB_DIM = 1024
NEG_SLOPE = 0.2


def _leaky(x):
    return jnp.where(x >= 0, x, NEG_SLOPE * x)


def _bf(x):
    return x.astype(jnp.bfloat16)


def _topk_body(xt_ref, xf_ref, xxc_ref, xxr_ref, idx_ref, *, T, N):
    xt = xt_ref[...]
    xf = xf_ref[...]
    s = lax.dot_general(_bf(xt), _bf(xf), (((1,), (1,)), ((), ())),
                        preferred_element_type=jnp.float32)
    d = 2.0 * s - xxr_ref[0] - xxc_ref[...]
    iota = lax.broadcasted_iota(jnp.int32, (T, N), 1)
    for j in range(KK):
        m = jnp.max(d, axis=1, keepdims=True)
        am = jnp.min(jnp.where(d >= m, iota, N), axis=1)
        idx_ref[0, j, :] = am
        d = jnp.where(iota == am[:, None], -jnp.inf, d)


def _topk(xt, xx, B, N, T):
    C = xt.shape[1]
    R = N // T
    body = functools.partial(_topk_body, T=T, N=N)
    return pl.pallas_call(
        body,
        grid=(B, R),
        in_specs=[
            pl.BlockSpec((T, C), lambda b, r: (b * R + r, 0)),
            pl.BlockSpec((N, C), lambda b, r: (b, 0)),
            pl.BlockSpec((T, 1), lambda b, r: (b * R + r, 0)),
            pl.BlockSpec((1, 1, N), lambda b, r: (b, 0, 0)),
        ],
        out_specs=pl.BlockSpec((1, KK, T), lambda b, r: (b, 0, r)),
        out_shape=jax.ShapeDtypeStruct((B, KK, N), jnp.int32),
    )(xt, xt, xx, xx.reshape(B, 1, N))


def _edgeconv_body(e_ref, x_ref, w1_ref, w2_ref, b_ref, g_ref, bt_ref,
                   out_ref, xx_ref, a_s):
    k = pl.program_id(1)

    @pl.when(k == 0)
    def _():
        a_s[...] = lax.dot_general(
            _bf(x_ref[...]), _bf(w2_ref[...]), (((1,), (0,)), ((), ())),
            preferred_element_type=jnp.float32) + b_ref[...]

    h = lax.dot_general(_bf(e_ref[...]), _bf(w1_ref[...]), (((1,), (0,)), ((), ())),
                        preferred_element_type=jnp.float32)
    v = _leaky(g_ref[...] * (a_s[...] + h) + bt_ref[...])
    prev = jnp.where(k == 0, -jnp.inf, out_ref[...])
    m = jnp.maximum(prev, v)
    out_ref[...] = m

    @pl.when(k == KK - 1)
    def _():
        xx_ref[...] = jnp.sum(m * m, axis=1, keepdims=True)


def _edgeconv(E, X, w1t, w2t, b, g, bt, BN, T):
    CE = E.shape[1]
    CX = X.shape[1]
    do = w1t.shape[1]
    Tn = BN // T
    return pl.pallas_call(
        _edgeconv_body,
        grid=(Tn, KK),
        in_specs=[
            pl.BlockSpec((T, CE), lambda t, k: (k * Tn + t, 0)),
            pl.BlockSpec((T, CX), lambda t, k: (t, 0)),
            pl.BlockSpec((CE, do), lambda t, k: (0, 0)),
            pl.BlockSpec((CX, do), lambda t, k: (0, 0)),
            pl.BlockSpec((1, do), lambda t, k: (0, 0)),
            pl.BlockSpec((1, do), lambda t, k: (0, 0)),
            pl.BlockSpec((1, do), lambda t, k: (0, 0)),
        ],
        out_specs=[
            pl.BlockSpec((T, do), lambda t, k: (t, 0)),
            pl.BlockSpec((T, 1), lambda t, k: (t, 0)),
        ],
        out_shape=[
            jax.ShapeDtypeStruct((BN, do), jnp.float32),
            jax.ShapeDtypeStruct((BN, 1), jnp.float32),
        ],
        scratch_shapes=[pltpu.VMEM((T, do), jnp.float32)],
    )(E, X, w1t, w2t, b, g, bt)


def _head_body(x1_ref, x2_ref, x3_ref, x4_ref, w1_ref, w2_ref, w3_ref, w4_ref,
               b_ref, g_ref, bt_ref, o_ref):
    r = pl.program_id(1)

    def dot(xr, wr):
        return lax.dot_general(_bf(xr[...]), _bf(wr[...]),
                               (((1,), (0,)), ((), ())),
                               preferred_element_type=jnp.float32)

    h = dot(x1_ref, w1_ref) + dot(x2_ref, w2_ref)
    h = h + dot(x3_ref, w3_ref) + dot(x4_ref, w4_ref)
    h = _leaky(g_ref[...] * (h + b_ref[...]) + bt_ref[...])
    pmax = jnp.max(h, axis=0, keepdims=True)
    psum = jnp.sum(h, axis=0, keepdims=True)

    @pl.when(r == 0)
    def _():
        o_ref[0, :, :] = jnp.concatenate([pmax, psum], axis=0)

    @pl.when(r > 0)
    def _():
        o_ref[0, 0, :] = jnp.maximum(o_ref[0, 0, :], pmax[0])
        o_ref[0, 1, :] = o_ref[0, 1, :] + psum[0]


def _head(xs, wfs, fb, fg, fbt, B, N, T):
    R = N // T
    in_specs = []
    args = []
    for x in xs:
        d = x.shape[1]
        in_specs.append(pl.BlockSpec((T, d), lambda b, r, R=R: (b * R + r, 0)))
        args.append(x)
    for w in wfs:
        d = w.shape[0]
        in_specs.append(pl.BlockSpec((d, EMB_DIM), lambda b, r: (0, 0)))
        args.append(w)
    for v in (fb, fg, fbt):
        in_specs.append(pl.BlockSpec((1, EMB_DIM), lambda b, r: (0, 0)))
        args.append(v)
    return pl.pallas_call(
        _head_body,
        grid=(B, R),
        in_specs=in_specs,
        out_specs=pl.BlockSpec((1, 2, EMB_DIM), lambda b, r: (b, 0, 0)),
        out_shape=jax.ShapeDtypeStruct((B, 2, EMB_DIM), jnp.float32),
    )(*args)


def _mlp_body(x_ref, w0_ref, b0_ref, g0_ref, t0_ref, w1_ref, b1_ref, g1_ref,
              t1_ref, w2_ref, b2_ref, o_ref):
    def dot(x, wr):
        return lax.dot_general(_bf(x), _bf(wr[...]), (((1,), (0,)), ((), ())),
                               preferred_element_type=jnp.float32)

    h = _leaky(g0_ref[...] * (dot(x_ref[...], w0_ref) + b0_ref[...]) + t0_ref[...])
    h = _leaky(g1_ref[...] * (dot(h, w1_ref) + b1_ref[...]) + t1_ref[...])
    o_ref[...] = dot(h, w2_ref) + b2_ref[...]


def _mlp(x, args):
    B = x.shape[0]
    n_out = args[-2].shape[1]
    return pl.pallas_call(
        _mlp_body,
        out_shape=jax.ShapeDtypeStruct((B, n_out), jnp.float32),
    )(x, *args)


def _gather_diff(X, idx_tr, B, N):
    BN, C = X.shape
    NW = 32
    PPW = BN // NW
    P = 128
    NCH = PPW // P
    mesh = plsc.VectorSubcoreMesh(core_axis_name="c", subcore_axis_name="s")

    @functools.partial(
        pl.kernel,
        out_type=jax.ShapeDtypeStruct((KK * BN, C), jnp.float32),
        mesh=mesh,
        scratch_types=[
            pltpu.VMEM((P,), jnp.int32),
            pltpu.VMEM((P, C), jnp.float32),
            pltpu.VMEM((P, C), jnp.float32),
            pltpu.SemaphoreType.DMA,
        ],
    )
    def gather_k(x_hbm, idx_hbm, e_hbm, idx_v, nbr_v, ctr_v, sem):
        wid = lax.axis_index("s") * 2 + lax.axis_index("c")
        p0 = wid * PPW
        b = p0 // N
        boff = b * N
        for ch in range(NCH):
            gp = p0 + ch * P
            pltpu.sync_copy(x_hbm.at[pl.ds(gp, P)], ctr_v)

            def k_step(k, _):
                pltpu.sync_copy(idx_hbm.at[b, k, pl.ds(gp - boff, P)], idx_v)
                for i in range(P // 16):
                    sl = pl.ds(i * 16, 16)
                    idx_v[sl] = idx_v[sl] + boff
                pltpu.async_copy(x_hbm.at[idx_v], nbr_v, sem).wait()

                def sub_row(n, _):
                    for c2 in range(C // 16):
                        s2 = pl.ds(c2 * 16, 16)
                        nbr_v[n, s2] = nbr_v[n, s2] - ctr_v[n, s2]
                    return 0

                lax.fori_loop(0, P, sub_row, 0)
                pltpu.sync_copy(nbr_v, e_hbm.at[pl.ds(k * BN + gp, P)])
                return 0

            lax.fori_loop(0, KK, k_step, 0)

    return gather_k(X, idx_tr)


def kernel(cloud, indices, params):
    B, N, _ = cloud.shape
    BN = B * N
    xt = cloud.reshape(BN, 3)
    xx = None
    idx_tr = jnp.transpose(indices, (0, 2, 1))

    xs = []
    for i in range(4):
        C = xt.shape[1]
        do = H_DIMS[i]
        if i > 0:
            idx_tr = _topk(xt, xx, B, N, T=256)
        w = params['conv_w_%d' % i]
        w1t = w[:, :C].T
        if C % 128:
            cp = 128 - C % 128
            xg = jnp.pad(xt, ((0, 0), (0, cp)))
            w1t = jnp.pad(w1t, ((0, cp), (0, 0)))
        else:
            xg = xt
        E = _gather_diff(xg, idx_tr, B, N)
        xt, xx = _edgeconv(
            E, xt, w1t, w[:, C:].T,
            params['conv_b_%d' % i][None, :], params['conv_g_%d' % i][None, :],
            params['conv_bt_%d' % i][None, :], BN, T=512)
        xs.append(xt)

    offs = [0, 64, 128, 256, 512]
    fw = params['final_w']
    wfs = [fw[:, offs[i]:offs[i + 1]].T for i in range(4)]
    pooled = _head(xs, wfs, params['final_b'][None, :], params['final_g'][None, :],
                   params['final_bt'][None, :], B, N, T=512)
    emb = jnp.concatenate([pooled[:, 0, :], pooled[:, 1, :] / N], axis=1)

    margs = []
    for j in range(2):
        margs += [params['mlp_w_%d' % j].T, params['mlp_b_%d' % j][None, :],
                  params['mlp_g_%d' % j][None, :], params['mlp_bt_%d' % j][None, :]]
    margs += [params['out_w'].T, params['out_b'][None, :]]
    return _mlp(emb, margs)


# speedup vs baseline: 8.4029x; 1.1363x over previous
import functools

import jax
import jax.numpy as jnp
from jax import lax
from jax.experimental import pallas as pl
from jax.experimental.pallas import tpu as pltpu
from jax.experimental.pallas import tpu_sc as plsc

KK = 20
H_DIMS = [64, 64, 128, 256]
EMB_DIM = 1024
NEG_SLOPE = 0.2


def _leaky(x):
    return jnp.where(x >= 0, x, NEG_SLOPE * x)


def _bf(x):
    return x.astype(jnp.bfloat16)


def _topk_body(xt_ref, xf_ref, xxc_ref, xxr_ref, idx_ref, *, T, N):
    xt = xt_ref[...]
    xf = xf_ref[...]
    s = lax.dot_general(_bf(xt), _bf(xf), (((1,), (1,)), ((), ())),
                        preferred_element_type=jnp.float32)
    d = 2.0 * s - xxr_ref[0] - xxc_ref[...]
    iota = lax.broadcasted_iota(jnp.int32, (T, N), 1)
    for j in range(KK):
        m = jnp.max(d, axis=1, keepdims=True)
        am = jnp.min(jnp.where(d >= m, iota, N), axis=1)
        idx_ref[0, j, :] = am
        d = jnp.where(iota == am[:, None], -jnp.inf, d)


def _topk(xt, xx, B, N, T):
    C = xt.shape[1]
    R = N // T
    body = functools.partial(_topk_body, T=T, N=N)
    return pl.pallas_call(
        body,
        grid=(B, R),
        in_specs=[
            pl.BlockSpec((T, C), lambda b, r: (b * R + r, 0)),
            pl.BlockSpec((N, C), lambda b, r: (b, 0)),
            pl.BlockSpec((T, 1), lambda b, r: (b * R + r, 0)),
            pl.BlockSpec((1, 1, N), lambda b, r: (b, 0, 0)),
        ],
        out_specs=pl.BlockSpec((1, KK, T), lambda b, r: (b, 0, r)),
        out_shape=jax.ShapeDtypeStruct((B, KK, N), jnp.int32),
    )(xt, xt, xx, xx.reshape(B, 1, N))


def _edgeconv_body(e_ref, x_ref, w1_ref, w2_ref, b_ref, g_ref, bt_ref,
                   out_ref, xx_ref, a_s):
    k = pl.program_id(1)

    @pl.when(k == 0)
    def _():
        a_s[...] = lax.dot_general(
            _bf(x_ref[...]), _bf(w2_ref[...]), (((1,), (0,)), ((), ())),
            preferred_element_type=jnp.float32) + b_ref[...]

    cx = x_ref.shape[1]
    diff = e_ref[:, :cx] - x_ref[...]
    h = lax.dot_general(_bf(diff), _bf(w1_ref[...]), (((1,), (0,)), ((), ())),
                        preferred_element_type=jnp.float32)
    v = _leaky(g_ref[...] * (a_s[...] + h) + bt_ref[...])
    prev = jnp.where(k == 0, -jnp.inf, out_ref[...])
    m = jnp.maximum(prev, v)
    out_ref[...] = m

    @pl.when(k == KK - 1)
    def _():
        xx_ref[...] = jnp.sum(m * m, axis=1, keepdims=True)


def _edgeconv(E, X, w1t, w2t, b, g, bt, BN, T):
    CE = E.shape[1]
    CX = X.shape[1]
    do = w1t.shape[1]
    Tn = BN // T
    return pl.pallas_call(
        _edgeconv_body,
        grid=(Tn, KK),
        in_specs=[
            pl.BlockSpec((T, CE), lambda t, k: (k * Tn + t, 0)),
            pl.BlockSpec((T, CX), lambda t, k: (t, 0)),
            pl.BlockSpec((CX, do), lambda t, k: (0, 0)),
            pl.BlockSpec((CX, do), lambda t, k: (0, 0)),
            pl.BlockSpec((1, do), lambda t, k: (0, 0)),
            pl.BlockSpec((1, do), lambda t, k: (0, 0)),
            pl.BlockSpec((1, do), lambda t, k: (0, 0)),
        ],
        out_specs=[
            pl.BlockSpec((T, do), lambda t, k: (t, 0)),
            pl.BlockSpec((T, 1), lambda t, k: (t, 0)),
        ],
        out_shape=[
            jax.ShapeDtypeStruct((BN, do), jnp.float32),
            jax.ShapeDtypeStruct((BN, 1), jnp.float32),
        ],
        scratch_shapes=[pltpu.VMEM((T, do), jnp.float32)],
    )(E, X, w1t, w2t, b, g, bt)


def _head_body(x1_ref, x2_ref, x3_ref, x4_ref, w1_ref, w2_ref, w3_ref, w4_ref,
               b_ref, g_ref, bt_ref, o_ref):
    r = pl.program_id(1)

    def dot(xr, wr):
        return lax.dot_general(_bf(xr[...]), _bf(wr[...]),
                               (((1,), (0,)), ((), ())),
                               preferred_element_type=jnp.float32)

    h = dot(x1_ref, w1_ref) + dot(x2_ref, w2_ref)
    h = h + dot(x3_ref, w3_ref) + dot(x4_ref, w4_ref)
    h = _leaky(g_ref[...] * (h + b_ref[...]) + bt_ref[...])
    pmax = jnp.max(h, axis=0, keepdims=True)
    psum = jnp.sum(h, axis=0, keepdims=True)

    @pl.when(r == 0)
    def _():
        o_ref[0, :, :] = jnp.concatenate([pmax, psum], axis=0)

    @pl.when(r > 0)
    def _():
        o_ref[0, 0, :] = jnp.maximum(o_ref[0, 0, :], pmax[0])
        o_ref[0, 1, :] = o_ref[0, 1, :] + psum[0]


def _head(xs, wfs, fb, fg, fbt, B, N, T):
    R = N // T
    in_specs = []
    args = []
    for x in xs:
        d = x.shape[1]
        in_specs.append(pl.BlockSpec((T, d), lambda b, r, R=R: (b * R + r, 0)))
        args.append(x)
    for w in wfs:
        d = w.shape[0]
        in_specs.append(pl.BlockSpec((d, EMB_DIM), lambda b, r: (0, 0)))
        args.append(w)
    for v in (fb, fg, fbt):
        in_specs.append(pl.BlockSpec((1, EMB_DIM), lambda b, r: (0, 0)))
        args.append(v)
    return pl.pallas_call(
        _head_body,
        grid=(B, R),
        in_specs=in_specs,
        out_specs=pl.BlockSpec((1, 2, EMB_DIM), lambda b, r: (b, 0, 0)),
        out_shape=jax.ShapeDtypeStruct((B, 2, EMB_DIM), jnp.float32),
    )(*args)


def _mlp_body(x_ref, w0_ref, b0_ref, g0_ref, t0_ref, w1_ref, b1_ref, g1_ref,
              t1_ref, w2_ref, b2_ref, o_ref):
    def dot(x, wr):
        return lax.dot_general(_bf(x), _bf(wr[...]), (((1,), (0,)), ((), ())),
                               preferred_element_type=jnp.float32)

    h = _leaky(g0_ref[...] * (dot(x_ref[...], w0_ref) + b0_ref[...]) + t0_ref[...])
    h = _leaky(g1_ref[...] * (dot(h, w1_ref) + b1_ref[...]) + t1_ref[...])
    o_ref[...] = dot(h, w2_ref) + b2_ref[...]


def _mlp(x, args):
    B = x.shape[0]
    n_out = args[-2].shape[1]
    return pl.pallas_call(
        _mlp_body,
        out_shape=jax.ShapeDtypeStruct((B, n_out), jnp.float32),
    )(x, *args)


def _gather_nbr(X, idx_tr, B, N):
    BN, C = X.shape
    NW = 32
    PPW = BN // NW
    P = 128
    NT = (PPW // P) * KK
    mesh = plsc.VectorSubcoreMesh(core_axis_name="c", subcore_axis_name="s")

    @functools.partial(
        pl.kernel,
        out_type=jax.ShapeDtypeStruct((KK * BN, C), jnp.float32),
        mesh=mesh,
        scratch_types=[
            pltpu.VMEM((2, P), jnp.int32),
            pltpu.VMEM((2, P, C), jnp.float32),
            pltpu.SemaphoreType.DMA((2,)),
            pltpu.SemaphoreType.DMA((2,)),
        ],
    )
    def gather_k(x_hbm, idx_hbm, e_hbm, idx_v, nbr_v, gsem, wsem):
        wid = lax.axis_index("s") * 2 + lax.axis_index("c")
        p0 = wid * PPW
        b = p0 // N
        boff = b * N

        def issue_gather(t, s):
            ch, k = t // KK, t % KK
            gp = p0 + ch * P
            pltpu.sync_copy(idx_hbm.at[b, k, pl.ds(gp - boff, P)], idx_v.at[s])
            for i in range(P // 16):
                sl = pl.ds(i * 16, 16)
                idx_v[s, sl] = idx_v[s, sl] + boff
            return pltpu.async_copy(x_hbm.at[idx_v.at[s]], nbr_v.at[s],
                                    gsem.at[s])

        def issue_write(t, s):
            ch, k = t // KK, t % KK
            gp = p0 + ch * P
            return pltpu.async_copy(nbr_v.at[s], e_hbm.at[pl.ds(k * BN + gp, P)],
                                    wsem.at[s])

        gathers = [None, None]
        writes = [None, None]
        for t in range(NT):
            s = t % 2
            if writes[s] is not None:
                writes[s].wait()
            gathers[s] = issue_gather(t, s)
            if t >= 1:
                sp = (t - 1) % 2
                gathers[sp].wait()
                writes[sp] = issue_write(t - 1, sp)
        sl = (NT - 1) % 2
        gathers[sl].wait()
        writes[sl] = issue_write(NT - 1, sl)
        writes[(NT - 2) % 2].wait()
        writes[sl].wait()

    return gather_k(X, idx_tr)


def kernel(cloud, indices, params):
    B, N, _ = cloud.shape
    BN = B * N
    xt = cloud.reshape(BN, 3)
    xx = None
    idx_tr = jnp.transpose(indices, (0, 2, 1))

    xs = []
    for i in range(4):
        C = xt.shape[1]
        do = H_DIMS[i]
        if i > 0:
            idx_tr = _topk(xt, xx, B, N, T=256)
        w = params['conv_w_%d' % i]
        if C % 128:
            xg = jnp.pad(xt, ((0, 0), (0, 128 - C % 128)))
        else:
            xg = xt
        E = _gather_nbr(xg, idx_tr, B, N)
        xt, xx = _edgeconv(
            E, xt, w[:, :C].T, w[:, C:].T,
            params['conv_b_%d' % i][None, :], params['conv_g_%d' % i][None, :],
            params['conv_bt_%d' % i][None, :], BN, T=512)
        xs.append(xt)

    offs = [0, 64, 128, 256, 512]
    fw = params['final_w']
    wfs = [fw[:, offs[i]:offs[i + 1]].T for i in range(4)]
    pooled = _head(xs, wfs, params['final_b'][None, :], params['final_g'][None, :],
                   params['final_bt'][None, :], B, N, T=512)
    emb = jnp.concatenate([pooled[:, 0, :], pooled[:, 1, :] / N], axis=1)

    margs = []
    for j in range(2):
        margs += [params['mlp_w_%d' % j].T, params['mlp_b_%d' % j][None, :],
                  params['mlp_g_%d' % j][None, :], params['mlp_bt_%d' % j][None, :]]
    margs += [params['out_w'].T, params['out_b'][None, :]]
    return _mlp(emb, margs)


# speedup vs baseline: 9.7862x; 1.1646x over previous
import functools

import jax
import jax.numpy as jnp
from jax import lax
from jax.experimental import pallas as pl
from jax.experimental.pallas import tpu as pltpu
from jax.experimental.pallas import tpu_sc as plsc

KK = 20
H_DIMS = [64, 64, 128, 256]
EMB_DIM = 1024
NEG_SLOPE = 0.2


def _leaky(x):
    return jnp.where(x >= 0, x, NEG_SLOPE * x)


def _bf(x):
    return x.astype(jnp.bfloat16)


def _topk_body(xt_ref, xf_ref, xxc_ref, xxr_ref, idx_ref, *, T, N):
    xt = xt_ref[...]
    xf = xf_ref[...]
    s = lax.dot_general(_bf(xt), _bf(xf), (((1,), (1,)), ((), ())),
                        preferred_element_type=jnp.float32)
    d = 2.0 * s - xxr_ref[0] - xxc_ref[...]
    iota = lax.broadcasted_iota(jnp.int32, (T, N), 1)
    for j in range(KK):
        m = jnp.max(d, axis=1, keepdims=True)
        am = jnp.min(jnp.where(d >= m, iota, N), axis=1)
        idx_ref[0, j, :] = am
        d = jnp.where(iota == am[:, None], -jnp.inf, d)


def _topk(xt, xx, B, N, T):
    C = xt.shape[1]
    R = N // T
    body = functools.partial(_topk_body, T=T, N=N)
    return pl.pallas_call(
        body,
        grid=(B, R),
        in_specs=[
            pl.BlockSpec((T, C), lambda b, r: (b * R + r, 0)),
            pl.BlockSpec((N, C), lambda b, r: (b, 0)),
            pl.BlockSpec((T, 1), lambda b, r: (b * R + r, 0)),
            pl.BlockSpec((1, 1, N), lambda b, r: (b, 0, 0)),
        ],
        out_specs=pl.BlockSpec((1, KK, T), lambda b, r: (b, 0, r)),
        out_shape=jax.ShapeDtypeStruct((B, KK, N), jnp.int32),
    )(xt, xt, xx, xx.reshape(B, 1, N))


def _edgeconv_body(e_ref, x_ref, w1_ref, w2_ref, b_ref, g_ref, bt_ref,
                   out_ref, xx_ref, a_s):
    k = pl.program_id(1)

    @pl.when(k == 0)
    def _():
        a_s[...] = lax.dot_general(
            _bf(x_ref[...]), _bf(w2_ref[...]), (((1,), (0,)), ((), ())),
            preferred_element_type=jnp.float32) + b_ref[...]

    cx = x_ref.shape[1]
    diff = e_ref[:, :cx] - x_ref[...]
    h = lax.dot_general(_bf(diff), _bf(w1_ref[...]), (((1,), (0,)), ((), ())),
                        preferred_element_type=jnp.float32)
    prev = jnp.where(k == 0, -jnp.inf, out_ref[...])
    m = jnp.maximum(prev, a_s[...] + h)

    @pl.when(k == KK - 1)
    def _():
        x = _leaky(g_ref[...] * m + bt_ref[...])
        out_ref[...] = x
        xx_ref[...] = jnp.sum(x * x, axis=1, keepdims=True)

    @pl.when(k < KK - 1)
    def _():
        out_ref[...] = m


def _edgeconv(E, X, w1t, w2t, b, g, bt, BN, T):
    CE = E.shape[1]
    CX = X.shape[1]
    do = w1t.shape[1]
    Tn = BN // T
    return pl.pallas_call(
        _edgeconv_body,
        grid=(Tn, KK),
        in_specs=[
            pl.BlockSpec((T, CE), lambda t, k: (k * Tn + t, 0)),
            pl.BlockSpec((T, CX), lambda t, k: (t, 0)),
            pl.BlockSpec((CX, do), lambda t, k: (0, 0)),
            pl.BlockSpec((CX, do), lambda t, k: (0, 0)),
            pl.BlockSpec((1, do), lambda t, k: (0, 0)),
            pl.BlockSpec((1, do), lambda t, k: (0, 0)),
            pl.BlockSpec((1, do), lambda t, k: (0, 0)),
        ],
        out_specs=[
            pl.BlockSpec((T, do), lambda t, k: (t, 0)),
            pl.BlockSpec((T, 1), lambda t, k: (t, 0)),
        ],
        out_shape=[
            jax.ShapeDtypeStruct((BN, do), jnp.float32),
            jax.ShapeDtypeStruct((BN, 1), jnp.float32),
        ],
        scratch_shapes=[pltpu.VMEM((T, do), jnp.float32)],
    )(E, X, w1t, w2t, b, g, bt)


def _head_body(x1_ref, x2_ref, x3_ref, x4_ref, w1_ref, w2_ref, w3_ref, w4_ref,
               b_ref, g_ref, bt_ref, o_ref):
    r = pl.program_id(1)

    def dot(xr, wr):
        return lax.dot_general(_bf(xr[...]), _bf(wr[...]),
                               (((1,), (0,)), ((), ())),
                               preferred_element_type=jnp.float32)

    h = dot(x1_ref, w1_ref) + dot(x2_ref, w2_ref)
    h = h + dot(x3_ref, w3_ref) + dot(x4_ref, w4_ref)
    h = _leaky(g_ref[...] * (h + b_ref[...]) + bt_ref[...])
    pmax = jnp.max(h, axis=0, keepdims=True)
    psum = jnp.sum(h, axis=0, keepdims=True)

    @pl.when(r == 0)
    def _():
        o_ref[0, :, :] = jnp.concatenate([pmax, psum], axis=0)

    @pl.when(r > 0)
    def _():
        o_ref[0, 0, :] = jnp.maximum(o_ref[0, 0, :], pmax[0])
        o_ref[0, 1, :] = o_ref[0, 1, :] + psum[0]


def _head(xs, wfs, fb, fg, fbt, B, N, T):
    R = N // T
    in_specs = []
    args = []
    for x in xs:
        d = x.shape[1]
        in_specs.append(pl.BlockSpec((T, d), lambda b, r, R=R: (b * R + r, 0)))
        args.append(x)
    for w in wfs:
        d = w.shape[0]
        in_specs.append(pl.BlockSpec((d, EMB_DIM), lambda b, r: (0, 0)))
        args.append(w)
    for v in (fb, fg, fbt):
        in_specs.append(pl.BlockSpec((1, EMB_DIM), lambda b, r: (0, 0)))
        args.append(v)
    return pl.pallas_call(
        _head_body,
        grid=(B, R),
        in_specs=in_specs,
        out_specs=pl.BlockSpec((1, 2, EMB_DIM), lambda b, r: (b, 0, 0)),
        out_shape=jax.ShapeDtypeStruct((B, 2, EMB_DIM), jnp.float32),
    )(*args)


def _mlp_body(x_ref, w0_ref, b0_ref, g0_ref, t0_ref, w1_ref, b1_ref, g1_ref,
              t1_ref, w2_ref, b2_ref, o_ref):
    def dot(x, wr):
        return lax.dot_general(_bf(x), _bf(wr[...]), (((1,), (0,)), ((), ())),
                               preferred_element_type=jnp.float32)

    h = _leaky(g0_ref[...] * (dot(x_ref[...], w0_ref) + b0_ref[...]) + t0_ref[...])
    h = _leaky(g1_ref[...] * (dot(h, w1_ref) + b1_ref[...]) + t1_ref[...])
    o_ref[...] = dot(h, w2_ref) + b2_ref[...]


def _mlp(x, args):
    B = x.shape[0]
    n_out = args[-2].shape[1]
    return pl.pallas_call(
        _mlp_body,
        out_shape=jax.ShapeDtypeStruct((B, n_out), jnp.float32),
    )(x, *args)


def _gather_nbr(X, idx_tr, B, N):
    BN, C = X.shape
    NW = 32
    PPW = BN // NW
    P = 128
    NT = (PPW // P) * KK
    mesh = plsc.VectorSubcoreMesh(core_axis_name="c", subcore_axis_name="s")

    @functools.partial(
        pl.kernel,
        out_type=jax.ShapeDtypeStruct((KK * BN, C), jnp.float32),
        mesh=mesh,
        scratch_types=[
            pltpu.VMEM((2, P), jnp.int32),
            pltpu.VMEM((2, P, C), jnp.float32),
            pltpu.SemaphoreType.DMA((2,)),
            pltpu.SemaphoreType.DMA((2,)),
        ],
    )
    def gather_k(x_hbm, idx_hbm, e_hbm, idx_v, nbr_v, gsem, wsem):
        wid = lax.axis_index("s") * 2 + lax.axis_index("c")
        p0 = wid * PPW
        b = p0 // N
        boff = b * N

        def issue_gather(t, s):
            ch, k = t // KK, t % KK
            gp = p0 + ch * P
            pltpu.sync_copy(idx_hbm.at[b, k, pl.ds(gp - boff, P)], idx_v.at[s])
            for i in range(P // 16):
                sl = pl.ds(i * 16, 16)
                idx_v[s, sl] = idx_v[s, sl] + boff
            return pltpu.async_copy(x_hbm.at[idx_v.at[s]], nbr_v.at[s],
                                    gsem.at[s])

        def issue_write(t, s):
            ch, k = t // KK, t % KK
            gp = p0 + ch * P
            return pltpu.async_copy(nbr_v.at[s], e_hbm.at[pl.ds(k * BN + gp, P)],
                                    wsem.at[s])

        gathers = [None, None]
        writes = [None, None]
        for t in range(NT):
            s = t % 2
            if writes[s] is not None:
                writes[s].wait()
            gathers[s] = issue_gather(t, s)
            if t >= 1:
                sp = (t - 1) % 2
                gathers[sp].wait()
                writes[sp] = issue_write(t - 1, sp)
        sl = (NT - 1) % 2
        gathers[sl].wait()
        writes[sl] = issue_write(NT - 1, sl)
        writes[(NT - 2) % 2].wait()
        writes[sl].wait()

    return gather_k(X, idx_tr)


def kernel(cloud, indices, params):
    B, N, _ = cloud.shape
    BN = B * N
    xt = cloud.reshape(BN, 3)
    xx = None
    idx_tr = jnp.transpose(indices, (0, 2, 1))

    xs = []
    for i in range(4):
        C = xt.shape[1]
        do = H_DIMS[i]
        if i > 0:
            idx_tr = _topk(xt, xx, B, N, T=256)
        w = params['conv_w_%d' % i]
        if C % 128:
            xg = jnp.pad(xt, ((0, 0), (0, 128 - C % 128)))
        else:
            xg = xt
        E = _gather_nbr(xg, idx_tr, B, N)
        xt, xx = _edgeconv(
            E, xt, w[:, :C].T, w[:, C:].T,
            params['conv_b_%d' % i][None, :], params['conv_g_%d' % i][None, :],
            params['conv_bt_%d' % i][None, :], BN, T=1024)
        xs.append(xt)

    offs = [0, 64, 128, 256, 512]
    fw = params['final_w']
    wfs = [fw[:, offs[i]:offs[i + 1]].T for i in range(4)]
    pooled = _head(xs, wfs, params['final_b'][None, :], params['final_g'][None, :],
                   params['final_bt'][None, :], B, N, T=512)
    emb = jnp.concatenate([pooled[:, 0, :], pooled[:, 1, :] / N], axis=1)

    margs = []
    for j in range(2):
        margs += [params['mlp_w_%d' % j].T, params['mlp_b_%d' % j][None, :],
                  params['mlp_g_%d' % j][None, :], params['mlp_bt_%d' % j][None, :]]
    margs += [params['out_w'].T, params['out_b'][None, :]]
    return _mlp(emb, margs)


# speedup vs baseline: 10.7705x; 1.1006x over previous
import functools

import jax
import jax.numpy as jnp
from jax import lax
from jax.experimental import pallas as pl
from jax.experimental.pallas import tpu as pltpu
from jax.experimental.pallas import tpu_sc as plsc

KK = 20
H_DIMS = [64, 64, 128, 256]
EMB_DIM = 1024
NEG_SLOPE = 0.2


def _leaky(x):
    return jnp.where(x >= 0, x, NEG_SLOPE * x)


def _bf(x):
    return x.astype(jnp.bfloat16)


def _topk_body(xt_ref, xf_ref, xxc_ref, xxr_ref, idx_ref, *, T, N):
    xt = xt_ref[...]
    xf = xf_ref[...]
    s = lax.dot_general(_bf(xt), _bf(xf), (((1,), (1,)), ((), ())),
                        preferred_element_type=jnp.float32)
    d = 2.0 * s - xxr_ref[0] - xxc_ref[...]
    iota = lax.broadcasted_iota(jnp.int32, (T, N), 1)
    for j in range(KK):
        m = jnp.max(d, axis=1, keepdims=True)
        ge = d >= m
        am = jnp.min(jnp.where(ge, iota, N), axis=1)
        idx_ref[0, j, :] = am
        d = jnp.where(ge, -jnp.inf, d)


def _topk(xt, xx, B, N, T):
    C = xt.shape[1]
    R = N // T
    body = functools.partial(_topk_body, T=T, N=N)
    return pl.pallas_call(
        body,
        grid=(B, R),
        in_specs=[
            pl.BlockSpec((T, C), lambda b, r: (b * R + r, 0)),
            pl.BlockSpec((N, C), lambda b, r: (b, 0)),
            pl.BlockSpec((T, 1), lambda b, r: (b * R + r, 0)),
            pl.BlockSpec((1, 1, N), lambda b, r: (b, 0, 0)),
        ],
        out_specs=pl.BlockSpec((1, KK, T), lambda b, r: (b, 0, r)),
        out_shape=jax.ShapeDtypeStruct((B, KK, N), jnp.int32),
    )(xt, xt, xx, xx.reshape(B, 1, N))


def _edgeconv_body(e_ref, x_ref, w1_ref, w2_ref, b_ref, g_ref, bt_ref,
                   out_ref, xx_ref, a_s):
    k = pl.program_id(1)

    @pl.when(k == 0)
    def _():
        a_s[...] = lax.dot_general(
            _bf(x_ref[...]), _bf(w2_ref[...]), (((1,), (0,)), ((), ())),
            preferred_element_type=jnp.float32) + b_ref[...]

    cx = x_ref.shape[1]
    diff = e_ref[:, :cx] - x_ref[...]
    h = lax.dot_general(_bf(diff), _bf(w1_ref[...]), (((1,), (0,)), ((), ())),
                        preferred_element_type=jnp.float32)
    prev = jnp.where(k == 0, -jnp.inf, out_ref[...])
    m = jnp.maximum(prev, a_s[...] + h)

    @pl.when(k == KK - 1)
    def _():
        x = _leaky(g_ref[...] * m + bt_ref[...])
        out_ref[...] = x
        xx_ref[...] = jnp.sum(x * x, axis=1, keepdims=True)

    @pl.when(k < KK - 1)
    def _():
        out_ref[...] = m


def _edgeconv(E, X, w1t, w2t, b, g, bt, BN, T):
    CE = E.shape[1]
    CX = X.shape[1]
    do = w1t.shape[1]
    Tn = BN // T
    return pl.pallas_call(
        _edgeconv_body,
        grid=(Tn, KK),
        in_specs=[
            pl.BlockSpec((T, CE), lambda t, k: (k * Tn + t, 0)),
            pl.BlockSpec((T, CX), lambda t, k: (t, 0)),
            pl.BlockSpec((CX, do), lambda t, k: (0, 0)),
            pl.BlockSpec((CX, do), lambda t, k: (0, 0)),
            pl.BlockSpec((1, do), lambda t, k: (0, 0)),
            pl.BlockSpec((1, do), lambda t, k: (0, 0)),
            pl.BlockSpec((1, do), lambda t, k: (0, 0)),
        ],
        out_specs=[
            pl.BlockSpec((T, do), lambda t, k: (t, 0)),
            pl.BlockSpec((T, 1), lambda t, k: (t, 0)),
        ],
        out_shape=[
            jax.ShapeDtypeStruct((BN, do), jnp.float32),
            jax.ShapeDtypeStruct((BN, 1), jnp.float32),
        ],
        scratch_shapes=[pltpu.VMEM((T, do), jnp.float32)],
    )(E, X, w1t, w2t, b, g, bt)


def _head_body(x1_ref, x2_ref, x3_ref, x4_ref, w1_ref, w2_ref, w3_ref, w4_ref,
               b_ref, g_ref, bt_ref, o_ref):
    r = pl.program_id(1)

    def dot(xr, wr):
        return lax.dot_general(_bf(xr[...]), _bf(wr[...]),
                               (((1,), (0,)), ((), ())),
                               preferred_element_type=jnp.float32)

    h = dot(x1_ref, w1_ref) + dot(x2_ref, w2_ref)
    h = h + dot(x3_ref, w3_ref) + dot(x4_ref, w4_ref)
    h = _leaky(g_ref[...] * (h + b_ref[...]) + bt_ref[...])
    pmax = jnp.max(h, axis=0, keepdims=True)
    psum = jnp.sum(h, axis=0, keepdims=True)

    @pl.when(r == 0)
    def _():
        o_ref[0, :, :] = jnp.concatenate([pmax, psum], axis=0)

    @pl.when(r > 0)
    def _():
        o_ref[0, 0, :] = jnp.maximum(o_ref[0, 0, :], pmax[0])
        o_ref[0, 1, :] = o_ref[0, 1, :] + psum[0]


def _head(xs, wfs, fb, fg, fbt, B, N, T):
    R = N // T
    in_specs = []
    args = []
    for x in xs:
        d = x.shape[1]
        in_specs.append(pl.BlockSpec((T, d), lambda b, r, R=R: (b * R + r, 0)))
        args.append(x)
    for w in wfs:
        d = w.shape[0]
        in_specs.append(pl.BlockSpec((d, EMB_DIM), lambda b, r: (0, 0)))
        args.append(w)
    for v in (fb, fg, fbt):
        in_specs.append(pl.BlockSpec((1, EMB_DIM), lambda b, r: (0, 0)))
        args.append(v)
    return pl.pallas_call(
        _head_body,
        grid=(B, R),
        in_specs=in_specs,
        out_specs=pl.BlockSpec((1, 2, EMB_DIM), lambda b, r: (b, 0, 0)),
        out_shape=jax.ShapeDtypeStruct((B, 2, EMB_DIM), jnp.float32),
    )(*args)


def _mlp_body(x_ref, w0_ref, b0_ref, g0_ref, t0_ref, w1_ref, b1_ref, g1_ref,
              t1_ref, w2_ref, b2_ref, o_ref):
    def dot(x, wr):
        return lax.dot_general(_bf(x), _bf(wr[...]), (((1,), (0,)), ((), ())),
                               preferred_element_type=jnp.float32)

    h = _leaky(g0_ref[...] * (dot(x_ref[...], w0_ref) + b0_ref[...]) + t0_ref[...])
    h = _leaky(g1_ref[...] * (dot(h, w1_ref) + b1_ref[...]) + t1_ref[...])
    o_ref[...] = dot(h, w2_ref) + b2_ref[...]


def _mlp(x, args):
    B = x.shape[0]
    n_out = args[-2].shape[1]
    return pl.pallas_call(
        _mlp_body,
        out_shape=jax.ShapeDtypeStruct((B, n_out), jnp.float32),
    )(x, *args)


def _gather_nbr(X, idx_tr, B, N):
    BN, C = X.shape
    NW = 32
    PPW = BN // NW
    P = 128
    NT = (PPW // P) * KK
    mesh = plsc.VectorSubcoreMesh(core_axis_name="c", subcore_axis_name="s")

    @functools.partial(
        pl.kernel,
        out_type=jax.ShapeDtypeStruct((KK * BN, C), jnp.float32),
        mesh=mesh,
        scratch_types=[
            pltpu.VMEM((2, P), jnp.int32),
            pltpu.VMEM((2, P, C), jnp.float32),
            pltpu.SemaphoreType.DMA((2,)),
            pltpu.SemaphoreType.DMA((2,)),
        ],
    )
    def gather_k(x_hbm, idx_hbm, e_hbm, idx_v, nbr_v, gsem, wsem):
        wid = lax.axis_index("s") * 2 + lax.axis_index("c")
        p0 = wid * PPW
        b = p0 // N
        boff = b * N

        def issue_gather(t, s):
            ch, k = t // KK, t % KK
            gp = p0 + ch * P
            pltpu.sync_copy(idx_hbm.at[b, k, pl.ds(gp - boff, P)], idx_v.at[s])
            for i in range(P // 16):
                sl = pl.ds(i * 16, 16)
                idx_v[s, sl] = idx_v[s, sl] + boff
            return pltpu.async_copy(x_hbm.at[idx_v.at[s]], nbr_v.at[s],
                                    gsem.at[s])

        def issue_write(t, s):
            ch, k = t // KK, t % KK
            gp = p0 + ch * P
            return pltpu.async_copy(nbr_v.at[s], e_hbm.at[pl.ds(k * BN + gp, P)],
                                    wsem.at[s])

        gathers = [None, None]
        writes = [None, None]
        for t in range(NT):
            s = t % 2
            if writes[s] is not None:
                writes[s].wait()
            gathers[s] = issue_gather(t, s)
            if t >= 1:
                sp = (t - 1) % 2
                gathers[sp].wait()
                writes[sp] = issue_write(t - 1, sp)
        sl = (NT - 1) % 2
        gathers[sl].wait()
        writes[sl] = issue_write(NT - 1, sl)
        writes[(NT - 2) % 2].wait()
        writes[sl].wait()

    return gather_k(X, idx_tr)


def kernel(cloud, indices, params):
    B, N, _ = cloud.shape
    BN = B * N
    xt = cloud.reshape(BN, 3)
    xx = None
    idx_tr = jnp.transpose(indices, (0, 2, 1))

    xs = []
    for i in range(4):
        C = xt.shape[1]
        do = H_DIMS[i]
        if i > 0:
            idx_tr = _topk(xt, xx, B, N, T=256)
        w = params['conv_w_%d' % i]
        if C % 128:
            xg = jnp.pad(xt, ((0, 0), (0, 128 - C % 128)))
        else:
            xg = xt
        E = _gather_nbr(xg, idx_tr, B, N)
        xt, xx = _edgeconv(
            E, xt, w[:, :C].T, w[:, C:].T,
            params['conv_b_%d' % i][None, :], params['conv_g_%d' % i][None, :],
            params['conv_bt_%d' % i][None, :], BN, T=1024)
        xs.append(xt)

    offs = [0, 64, 128, 256, 512]
    fw = params['final_w']
    wfs = [fw[:, offs[i]:offs[i + 1]].T for i in range(4)]
    pooled = _head(xs, wfs, params['final_b'][None, :], params['final_g'][None, :],
                   params['final_bt'][None, :], B, N, T=512)
    emb = jnp.concatenate([pooled[:, 0, :], pooled[:, 1, :] / N], axis=1)

    margs = []
    for j in range(2):
        margs += [params['mlp_w_%d' % j].T, params['mlp_b_%d' % j][None, :],
                  params['mlp_g_%d' % j][None, :], params['mlp_bt_%d' % j][None, :]]
    margs += [params['out_w'].T, params['out_b'][None, :]]
    return _mlp(emb, margs)
